# Initial kernel scaffold; baseline (speedup 1.0000x reference)
import jax
import jax.numpy as jnp
from jax.experimental import pallas as pl


def kernel(x, W, b, slices):
    raise NotImplementedError("write your pallas kernel here")


# speedup vs baseline: 2.1726x; 2.1726x over previous
import functools

import jax
import jax.numpy as jnp
from jax import lax
from jax.experimental import pallas as pl
from jax.experimental.pallas import tpu as pltpu
from jax.experimental.pallas import tpu_sc as plsc

L = 16
NC = 2
NS = 16
R = 128


def _sc_pooling(Bn, T, D):
    n = T // Bn
    half = n // 2
    NB = half // R
    DV = D // L
    nseg_per_core = Bn // NC

    mesh = plsc.VectorSubcoreMesh(
        core_axis_name="c", subcore_axis_name="s", num_cores=NC,
        num_subcores=NS)

    @functools.partial(
        pl.kernel,
        out_type=jax.ShapeDtypeStruct((Bn, D), jnp.float32),
        mesh=mesh,
        compiler_params=pltpu.CompilerParams(needs_layout_passes=False),
        scratch_types=[
            pltpu.VMEM((D + 2 * L,), jnp.float32),
            pltpu.VMEM((2, R, D), jnp.float32),
            pltpu.VMEM((R,), jnp.float32),
            pltpu.VMEM((D + L,), jnp.float32),
            pltpu.VMEM((D + L,), jnp.float32),
            pltpu.VMEM((D,), jnp.float32),
            pltpu.VMEM_SHARED((NS * 512,), jnp.float32),
            pltpu.SemaphoreType.DMA,
            pltpu.SemaphoreType.DMA,
        ],
    )
    def call(x_hbm, params_hbm, out_hbm, params_v, xbuf, logits_v,
             acc_ref, pacc_v, outbuf, shex, sem0, sem1):
        c = lax.axis_index("c")
        s = lax.axis_index("s")
        seg = c * nseg_per_core + s // 2
        h = s % 2
        base = seg * n + h * half

        pltpu.sync_copy(params_hbm, params_v)
        wxv = [params_v[pl.ds(L * j, L)] for j in range(DV)]
        lane = lax.iota(jnp.int32, L)
        pcoef_v = params_v[pl.ds(D, L)]
        pcoef = jnp.sum(jnp.where(lane == seg, pcoef_v, 0.0))
        bias = params_v[pl.ds(D + Bn, L)][0]
        lanef = lane.astype(jnp.float32)
        zero16 = jnp.zeros((L,), jnp.float32)
        for j in range(DV):
            acc_ref[pl.ds(L * j, L)] = zero16

        sems = (sem0, sem1)
        pltpu.async_copy(x_hbm.at[pl.ds(base, R), :], xbuf.at[0], sem0)
        pltpu.async_copy(x_hbm.at[pl.ds(base + R, R), :], xbuf.at[1], sem1)

        def process(blk, b, m, s_v):
            xb = xbuf.at[b]
            toff = (h * half + blk * R).astype(jnp.float32)

            def grp_a(g, bm_v):
                rb = g * L
                dv = zero16
                for r in range(L):
                    p = xb[rb + r, pl.ds(0, L)] * wxv[0]
                    for j in range(1, DV):
                        p = p + xb[rb + r, pl.ds(L * j, L)] * wxv[j]
                    dv = jnp.where(lane == r, jnp.sum(p), dv)
                tloc = lanef + (toff + (g * L).astype(jnp.float32))
                lv = dv + tloc * pcoef + bias
                logits_v[pl.ds(g * L, L)] = lv
                return jnp.maximum(bm_v, lv)

            bmax_v = lax.fori_loop(0, R // L, grp_a,
                                   jnp.full((L,), -1e30, jnp.float32))
            m_new = jnp.maximum(m, jnp.max(bmax_v))
            resc = jnp.exp(jnp.full((L,), m - m_new))
            s_v = s_v * resc
            for j in range(DV):
                acc_ref[pl.ds(L * j, L)] = acc_ref[pl.ds(L * j, L)] * resc

            def grp_b(g, sv):
                rb = g * L
                ev = jnp.exp(logits_v[pl.ds(g * L, L)] - m_new)
                for r in range(L):
                    esp = jnp.full((L,), ev[r])
                    for j in range(DV):
                        plsc.addupdate(
                            acc_ref.at[pl.ds(L * j, L)],
                            esp * xb[rb + r, pl.ds(L * j, L)])
                return sv + ev

            s_v = lax.fori_loop(0, R // L, grp_b, s_v)
            return m_new, s_v

        def pair(p, carry):
            m, s_v = carry
            for sub in range(2):
                blk = 2 * p + sub
                wait_src = x_hbm.at[pl.ds(0, R), :]
                pltpu.make_async_copy(wait_src, xbuf.at[sub],
                                      sems[sub]).wait()
                m, s_v = process(blk, sub, m, s_v)

                @pl.when(blk + 2 < NB)
                def _():
                    pltpu.async_copy(
                        x_hbm.at[pl.ds(base + (blk + 2) * R, R), :],
                        xbuf.at[sub], sems[sub])
            return m, s_v

        m, s_v = lax.fori_loop(
            0, NB // 2, pair,
            (jnp.float32(-1e30), zero16))

        s_loc = jnp.sum(s_v)
        st = jnp.where(lane == 0, m, jnp.where(lane == 1, s_loc, 0.0))
        acc_ref[pl.ds(D, L)] = st
        pltpu.sync_copy(acc_ref, shex.at[pl.ds(s * 512, D + L)])
        plsc.subcore_barrier()

        @pl.when(h == 0)
        def _():
            pltpu.sync_copy(shex.at[pl.ds((s + 1) * 512, D + L)], pacc_v)
            pst = pacc_v[pl.ds(D, L)]
            m2 = pst[0]
            s2 = pst[1]
            mf = jnp.maximum(m, m2)
            a1 = jnp.exp(jnp.full((L,), m - mf))
            a2 = jnp.exp(jnp.full((L,), m2 - mf))
            inv = 1.0 / (a1 * s_loc + a2 * s2)
            for j in range(DV):
                outbuf[pl.ds(L * j, L)] = (
                    acc_ref[pl.ds(L * j, L)] * a1
                    + pacc_v[pl.ds(L * j, L)] * a2) * inv
            pltpu.sync_copy(outbuf, out_hbm.at[seg])

    return call


def kernel(x, W, b, slices):
    T, D = x.shape
    Bn = slices.shape[0]
    wx = W[0, 1:]
    pcoef = W[0, 0] / slices.astype(jnp.float32)
    pad = jnp.zeros((2 * L - Bn - 1,), jnp.float32)
    params = jnp.concatenate([wx, pcoef, b.astype(jnp.float32), pad])
    return _sc_pooling(Bn, T, D)(x, params)


# speedup vs baseline: 3.7460x; 1.7242x over previous
import functools

import jax
import jax.numpy as jnp
from jax import lax
from jax.experimental import pallas as pl
from jax.experimental.pallas import tpu as pltpu
from jax.experimental.pallas import tpu_sc as plsc

L = 16
NC = 2
NS = 16
R = 128


def _sc_pooling(Bn, T, D):
    n = T // Bn
    half = n // 2
    NB = half // R
    DV = D // L
    nseg_per_core = Bn // NC

    mesh = plsc.VectorSubcoreMesh(
        core_axis_name="c", subcore_axis_name="s", num_cores=NC,
        num_subcores=NS)

    @functools.partial(
        pl.kernel,
        out_type=jax.ShapeDtypeStruct((Bn, D), jnp.float32),
        mesh=mesh,
        compiler_params=pltpu.CompilerParams(needs_layout_passes=False),
        scratch_types=[
            pltpu.VMEM((D + 2 * L,), jnp.float32),
            pltpu.VMEM((2, R, D), jnp.float32),
            pltpu.VMEM((R,), jnp.float32),
            pltpu.VMEM((D + L,), jnp.float32),
            pltpu.VMEM((D + L,), jnp.float32),
            pltpu.VMEM((D,), jnp.float32),
            pltpu.VMEM_SHARED((NS * 512,), jnp.float32),
            pltpu.SemaphoreType.DMA,
            pltpu.SemaphoreType.DMA,
        ],
    )
    def call(x_hbm, params_hbm, out_hbm, params_v, xbuf, logits_v,
             acc_ref, pacc_v, outbuf, shex, sem0, sem1):
        c = lax.axis_index("c")
        s = lax.axis_index("s")
        seg = c * nseg_per_core + s // 2
        h = s % 2
        base = seg * n + h * half

        pltpu.sync_copy(params_hbm, params_v)
        wxv = [params_v[pl.ds(L * j, L)] for j in range(DV)]
        lane = lax.iota(jnp.int32, L)
        pcoef_v = params_v[pl.ds(D, L)]
        pcoef = jnp.sum(jnp.where(lane == seg, pcoef_v, 0.0))
        bias = params_v[pl.ds(D + Bn, L)][0]
        lanef = lane.astype(jnp.float32)
        zero16 = jnp.zeros((L,), jnp.float32)
        for j in range(DV):
            acc_ref[pl.ds(L * j, L)] = zero16

        sems = (sem0, sem1)
        pltpu.async_copy(x_hbm.at[pl.ds(base, R), :], xbuf.at[0], sem0)
        pltpu.async_copy(x_hbm.at[pl.ds(base + R, R), :], xbuf.at[1], sem1)

        def process(blk, b, m, s_v):
            xb = xbuf.at[b]
            toff = (h * half + blk * R).astype(jnp.float32)

            def grp_a(g, bm_v):
                rb = g * L
                dv = zero16
                for q in range(0, L, 4):
                    p = [zero16] * 4
                    for j in range(DV):
                        w = wxv[j]
                        for r in range(4):
                            p[r] = p[r] + xb[rb + q + r,
                                             pl.ds(L * j, L)] * w
                    for r in range(4):
                        dv = jnp.where(lane == q + r, jnp.sum(p[r]), dv)
                tloc = lanef + (toff + (g * L).astype(jnp.float32))
                lv = dv + tloc * pcoef + bias
                logits_v[pl.ds(g * L, L)] = lv
                return jnp.maximum(bm_v, lv)

            bmax_v = lax.fori_loop(0, R // L, grp_a,
                                   jnp.full((L,), -1e30, jnp.float32))
            m_new = jnp.maximum(m, jnp.max(bmax_v))
            resc = jnp.exp(jnp.full((L,), m - m_new))
            s_v = s_v * resc
            for j in range(DV):
                acc_ref[pl.ds(L * j, L)] = acc_ref[pl.ds(L * j, L)] * resc

            def grp_b(g, sv):
                rb = g * L
                ev = jnp.exp(logits_v[pl.ds(g * L, L)] - m_new)
                for q in range(0, L, 8):
                    esp = [jnp.full((L,), ev[q + r]) for r in range(8)]
                    for j in range(DV):
                        t0 = acc_ref[pl.ds(L * j, L)]
                        t1 = esp[1] * xb[rb + q + 1, pl.ds(L * j, L)]
                        for r in range(0, 8, 2):
                            t0 = t0 + esp[r] * xb[rb + q + r,
                                                  pl.ds(L * j, L)]
                        for r in range(3, 8, 2):
                            t1 = t1 + esp[r] * xb[rb + q + r,
                                                  pl.ds(L * j, L)]
                        acc_ref[pl.ds(L * j, L)] = t0 + t1
                return sv + ev

            s_v = lax.fori_loop(0, R // L, grp_b, s_v)
            return m_new, s_v

        def pair(p, carry):
            m, s_v = carry
            for sub in range(2):
                blk = 2 * p + sub
                wait_src = x_hbm.at[pl.ds(0, R), :]
                pltpu.make_async_copy(wait_src, xbuf.at[sub],
                                      sems[sub]).wait()
                m, s_v = process(blk, sub, m, s_v)

                @pl.when(blk + 2 < NB)
                def _():
                    pltpu.async_copy(
                        x_hbm.at[pl.ds(base + (blk + 2) * R, R), :],
                        xbuf.at[sub], sems[sub])
            return m, s_v

        m, s_v = lax.fori_loop(
            0, NB // 2, pair,
            (jnp.float32(-1e30), zero16))

        s_loc = jnp.sum(s_v)
        st = jnp.where(lane == 0, m, jnp.where(lane == 1, s_loc, 0.0))
        acc_ref[pl.ds(D, L)] = st
        pltpu.sync_copy(acc_ref, shex.at[pl.ds(s * 512, D + L)])
        plsc.subcore_barrier()

        @pl.when(h == 0)
        def _():
            pltpu.sync_copy(shex.at[pl.ds((s + 1) * 512, D + L)], pacc_v)
            pst = pacc_v[pl.ds(D, L)]
            m2 = pst[0]
            s2 = pst[1]
            mf = jnp.maximum(m, m2)
            a1 = jnp.exp(jnp.full((L,), m - mf))
            a2 = jnp.exp(jnp.full((L,), m2 - mf))
            inv = 1.0 / (a1 * s_loc + a2 * s2)
            for j in range(DV):
                outbuf[pl.ds(L * j, L)] = (
                    acc_ref[pl.ds(L * j, L)] * a1
                    + pacc_v[pl.ds(L * j, L)] * a2) * inv
            pltpu.sync_copy(outbuf, out_hbm.at[seg])

    return call


def kernel(x, W, b, slices):
    T, D = x.shape
    Bn = slices.shape[0]
    wx = W[0, 1:]
    pcoef = W[0, 0] / slices.astype(jnp.float32)
    pad = jnp.zeros((2 * L - Bn - 1,), jnp.float32)
    params = jnp.concatenate([wx, pcoef, b.astype(jnp.float32), pad])
    return _sc_pooling(Bn, T, D)(x, params)


# speedup vs baseline: 5.3180x; 1.4197x over previous
import functools

import jax
import jax.numpy as jnp
from jax import lax
from jax.experimental import pallas as pl
from jax.experimental.pallas import tpu as pltpu
from jax.experimental.pallas import tpu_sc as plsc

L = 16
NC = 2
NS = 16
R = 128


def _sc_pooling(Bn, T, D):
    n = T // Bn
    half = n // 2
    NB = half // R
    DV = D // L
    nseg_per_core = Bn // NC

    mesh = plsc.VectorSubcoreMesh(
        core_axis_name="c", subcore_axis_name="s", num_cores=NC,
        num_subcores=NS)

    @functools.partial(
        pl.kernel,
        out_type=jax.ShapeDtypeStruct((Bn, D), jnp.float32),
        mesh=mesh,
        compiler_params=pltpu.CompilerParams(needs_layout_passes=False),
        scratch_types=[
            pltpu.VMEM((D + 2 * L,), jnp.float32),
            pltpu.VMEM((2, R, D), jnp.float32),
            pltpu.VMEM((R,), jnp.float32),
            pltpu.VMEM((D + L,), jnp.float32),
            pltpu.VMEM((D + L,), jnp.float32),
            pltpu.VMEM((D,), jnp.float32),
            pltpu.VMEM_SHARED((NS * 512,), jnp.float32),
            pltpu.SemaphoreType.DMA,
            pltpu.SemaphoreType.DMA,
        ],
    )
    def call(x_hbm, params_hbm, out_hbm, params_v, xbuf, logits_v,
             acc_ref, pacc_v, outbuf, shex, sem0, sem1):
        c = lax.axis_index("c")
        s = lax.axis_index("s")
        seg = c * nseg_per_core + s // 2
        h = s % 2
        base = seg * n + h * half

        pltpu.sync_copy(params_hbm, params_v)
        lane = lax.iota(jnp.int32, L)
        pcoef_v = params_v[pl.ds(D, L)]
        pcoef = jnp.sum(jnp.where(lane == seg, pcoef_v, 0.0))
        bias = params_v[pl.ds(D + Bn, L)][0]
        lanef = lane.astype(jnp.float32)
        zero16 = jnp.zeros((L,), jnp.float32)
        for j in range(DV):
            acc_ref[pl.ds(L * j, L)] = zero16

        sems = (sem0, sem1)
        pltpu.async_copy(x_hbm.at[pl.ds(base, R), :], xbuf.at[0], sem0)
        pltpu.async_copy(x_hbm.at[pl.ds(base + R, R), :], xbuf.at[1], sem1)

        def process(blk, b, m, s_v):
            xb = xbuf.at[b]
            toff = (h * half + blk * R).astype(jnp.float32)

            def grp_a(g, bm_v):
                rb = g * L
                dv = zero16
                for q in range(0, L, 8):
                    p = [zero16] * 8
                    for j in range(DV):
                        w = params_v[pl.ds(L * j, L)]
                        for r in range(8):
                            p[r] = p[r] + xb[rb + q + r,
                                             pl.ds(L * j, L)] * w
                    for r in range(8):
                        dv = jnp.where(lane == q + r, jnp.sum(p[r]), dv)
                tloc = lanef + (toff + (g * L).astype(jnp.float32))
                lv = dv + tloc * pcoef + bias
                logits_v[pl.ds(g * L, L)] = lv
                return jnp.maximum(bm_v, lv)

            bmax_v = lax.fori_loop(0, R // L, grp_a,
                                   jnp.full((L,), -1e30, jnp.float32))
            m_new = jnp.maximum(m, jnp.max(bmax_v))
            resc = jnp.exp(jnp.full((L,), m - m_new))
            s_v = s_v * resc
            for j in range(DV):
                acc_ref[pl.ds(L * j, L)] = acc_ref[pl.ds(L * j, L)] * resc

            def grp_b(g, sv):
                rb = g * L
                ev = jnp.exp(logits_v[pl.ds(g * L, L)] - m_new)
                for q in range(0, L, 8):
                    esp = [jnp.full((L,), ev[q + r]) for r in range(8)]
                    for jc in range(0, DV, 8):
                        contrib = []
                        for j in range(jc, jc + 8):
                            pr = [esp[r] * xb[rb + q + r, pl.ds(L * j, L)]
                                  for r in range(8)]
                            s01 = pr[0] + pr[1]
                            s23 = pr[2] + pr[3]
                            s45 = pr[4] + pr[5]
                            s67 = pr[6] + pr[7]
                            contrib.append((s01 + s23) + (s45 + s67))
                        for j in range(jc, jc + 8):
                            plsc.addupdate(acc_ref.at[pl.ds(L * j, L)],
                                           contrib[j - jc])
                return sv + ev

            s_v = lax.fori_loop(0, R // L, grp_b, s_v)
            return m_new, s_v

        def pair(p, carry):
            m, s_v = carry
            for sub in range(2):
                blk = 2 * p + sub
                wait_src = x_hbm.at[pl.ds(0, R), :]
                pltpu.make_async_copy(wait_src, xbuf.at[sub],
                                      sems[sub]).wait()
                m, s_v = process(blk, sub, m, s_v)

                @pl.when(blk + 2 < NB)
                def _():
                    pltpu.async_copy(
                        x_hbm.at[pl.ds(base + (blk + 2) * R, R), :],
                        xbuf.at[sub], sems[sub])
            return m, s_v

        m, s_v = lax.fori_loop(
            0, NB // 2, pair,
            (jnp.float32(-1e30), zero16))

        s_loc = jnp.sum(s_v)
        st = jnp.where(lane == 0, m, jnp.where(lane == 1, s_loc, 0.0))
        acc_ref[pl.ds(D, L)] = st
        pltpu.sync_copy(acc_ref, shex.at[pl.ds(s * 512, D + L)])
        plsc.subcore_barrier()

        @pl.when(h == 0)
        def _():
            pltpu.sync_copy(shex.at[pl.ds((s + 1) * 512, D + L)], pacc_v)
            pst = pacc_v[pl.ds(D, L)]
            m2 = pst[0]
            s2 = pst[1]
            mf = jnp.maximum(m, m2)
            a1 = jnp.exp(jnp.full((L,), m - mf))
            a2 = jnp.exp(jnp.full((L,), m2 - mf))
            inv = 1.0 / (a1 * s_loc + a2 * s2)
            for j in range(DV):
                outbuf[pl.ds(L * j, L)] = (
                    acc_ref[pl.ds(L * j, L)] * a1
                    + pacc_v[pl.ds(L * j, L)] * a2) * inv
            pltpu.sync_copy(outbuf, out_hbm.at[seg])

    return call


def kernel(x, W, b, slices):
    T, D = x.shape
    Bn = slices.shape[0]
    wx = W[0, 1:]
    pcoef = W[0, 0] / slices.astype(jnp.float32)
    pad = jnp.zeros((2 * L - Bn - 1,), jnp.float32)
    params = jnp.concatenate([wx, pcoef, b.astype(jnp.float32), pad])
    return _sc_pooling(Bn, T, D)(x, params)
